# Initial kernel scaffold; baseline (speedup 1.0000x reference)
import jax
import jax.numpy as jnp
from jax.experimental import pallas as pl


def kernel(z_e_flat, embedding_weight):
    raise NotImplementedError("write your pallas kernel here")


# speedup vs baseline: 1.4115x; 1.4115x over previous
import functools

import jax
import jax.numpy as jnp
from jax.experimental import pallas as pl
from jax.experimental.pallas import tpu as pltpu

_BR = 512


def _vq_block(z_ref, e_ref, zq_ref, idx_ref, loss_ref):
    i = pl.program_id(0)
    z = z_ref[...]
    e = e_ref[...]
    z_sq = jnp.sum(jnp.square(z), axis=1, keepdims=True)
    e_sq = jnp.sum(jnp.square(e), axis=1)
    z_dot_e = jax.lax.dot_general(
        z, e, (((1,), (1,)), ((), ())),
        preferred_element_type=jnp.float32)
    dist = z_sq - 2.0 * z_dot_e + e_sq[None, :]
    idx = jnp.argmin(dist, axis=1).astype(jnp.int32)
    idx_ref[0, 0, :] = idx
    k_iota = jax.lax.broadcasted_iota(jnp.int32, dist.shape, 1)
    onehot = (k_iota == idx[:, None]).astype(jnp.float32)
    z_q = jax.lax.dot_general(
        onehot, e, (((1,), (0,)), ((), ())),
        preferred_element_type=jnp.float32)
    zq_ref[...] = z_q
    part = jnp.sum(jnp.square(z - z_q))

    @pl.when(i == 0)
    def _init():
        loss_ref[0, 0] = part

    @pl.when(i != 0)
    def _acc():
        loss_ref[0, 0] += part


@jax.jit
def kernel(z_e_flat, embedding_weight):
    z_e_flat = z_e_flat.astype(jnp.float32)
    B, N, D = z_e_flat.shape
    K = embedding_weight.shape[0]
    M = B * N
    z_flat = z_e_flat.reshape(M, D)
    nblk = M // _BR

    zq, idx3, loss = pl.pallas_call(
        _vq_block,
        grid=(nblk,),
        in_specs=[
            pl.BlockSpec((_BR, D), lambda i: (i, 0)),
            pl.BlockSpec((K, D), lambda i: (0, 0)),
        ],
        out_specs=[
            pl.BlockSpec((_BR, D), lambda i: (i, 0)),
            pl.BlockSpec((1, 1, _BR), lambda i: (i, 0, 0)),
            pl.BlockSpec(memory_space=pltpu.SMEM, block_shape=(1, 1),
                         index_map=lambda i: (0, 0)),
        ],
        out_shape=[
            jax.ShapeDtypeStruct((M, D), jnp.float32),
            jax.ShapeDtypeStruct((nblk, 1, _BR), jnp.int32),
            jax.ShapeDtypeStruct((1, 1), jnp.float32),
        ],
    )(z_flat, embedding_weight)

    z_q = zq.reshape(B, N, D)
    indices = idx3.reshape(B, N)
    commit_loss = loss[0, 0] * (0.25 / (M * D))
    return (z_q, indices, commit_loss)
